# Initial kernel scaffold; baseline (speedup 1.0000x reference)
import functools

import jax
import jax.numpy as jnp
from jax import lax
from jax.experimental import pallas as pl
from jax.experimental.pallas import tpu as pltpu
from jax.experimental.pallas import tpu_sc as plsc

VOCAB = 100000
D = 64
B = 4096
L = 50
R = 128

N_TOK = B * L
CHUNK = 128
LANES = 16

_info = plsc.get_sparse_core_info()
NC, NS = _info.num_cores, _info.num_subcores
NW = NC * NS
TOK_PER_W = N_TOK // NW
NCHUNK = TOK_PER_W // CHUNK

_BS_STEPS = (64, 32, 16, 8, 4, 2, 1)

_mesh = plsc.VectorSubcoreMesh(core_axis_name="c", subcore_axis_name="s")


@functools.partial(
    pl.kernel,
    mesh=_mesh,
    out_type=jax.ShapeDtypeStruct((N_TOK, D), jnp.float32),
    scratch_types=[
        pltpu.VMEM((CHUNK,), jnp.int32),
        pltpu.VMEM((CHUNK, D), jnp.float32),
        pltpu.VMEM((R,), jnp.int32),
        pltpu.VMEM((R, D), jnp.float32),
        pltpu.VMEM((LANES,), jnp.int32),
        pltpu.VMEM((LANES,), jnp.int32),
        pltpu.SemaphoreType.DMA,
    ],
)
def _sc_lookup(x_hbm, pre_hbm, res_hbm, ridx_hbm, out_hbm,
               idx_v, rows_v, ridx_v, rtab_v, tpos_v, tslot_v, sem):
    cid = lax.axis_index("c")
    sid = lax.axis_index("s")
    wid = sid * NC + cid
    base = wid * TOK_PER_W

    pltpu.sync_copy(ridx_hbm, ridx_v)
    pltpu.sync_copy(res_hbm, rtab_v)

    iota16 = lax.iota(jnp.int32, LANES)
    zero16 = jnp.zeros((LANES,), jnp.int32)
    rmin_s = jnp.min(plsc.load_gather(ridx_v, [zero16]))
    rmax_s = jnp.max(plsc.load_gather(ridx_v, [zero16 + (R - 1)]))

    def chunk_body(ci, carry):
        tb = base + ci * CHUNK
        pltpu.sync_copy(x_hbm.at[pl.ds(tb, CHUNK)], idx_v)
        pltpu.async_copy(pre_hbm.at[idx_v], rows_v, sem).wait()

        mn = jnp.full((LANES,), jnp.iinfo(jnp.int32).max, jnp.int32)
        mx = jnp.full((LANES,), jnp.iinfo(jnp.int32).min, jnp.int32)
        for i in range(CHUNK // LANES):
            v = idx_v[pl.ds(i * LANES, LANES)]
            mn = jnp.minimum(mn, v)
            mx = jnp.maximum(mx, v)
        maybe = (jnp.min(mn) <= rmax_s) & (jnp.max(mx) >= rmin_s)

        def fixup():
            for i in range(CHUNK // LANES):
                v = idx_v[pl.ds(i * LANES, LANES)]
                lo = zero16
                for sz in _BS_STEPS:
                    mid = lo + sz
                    probe = plsc.load_gather(ridx_v, [mid - 1])
                    lo = jnp.where(probe < v, mid, lo)
                safe = jnp.minimum(lo, R - 1)
                hit = (plsc.load_gather(ridx_v, [safe]) == v) & (lo < R)
                cnt = jnp.sum(hit.astype(jnp.int32))

                def do_fix():
                    plsc.store_compressed(tpos_v, i * LANES + iota16, hit)
                    plsc.store_compressed(tslot_v, lo, hit)

                    def fix_one(j, c):
                        jb = jnp.full((LANES,), j, jnp.int32)
                        p_b = plsc.load_gather(tpos_v, [jb])
                        s_b = plsc.load_gather(tslot_v, [jb])
                        for d in range(D // LANES):
                            colv = iota16 + d * LANES
                            vals = plsc.load_gather(rtab_v, [s_b, colv])
                            plsc.store_scatter(rows_v, [p_b, colv], vals)
                        return c

                    lax.fori_loop(0, cnt, fix_one, 0)

                lax.cond(cnt > 0, do_fix, lambda: None)

        lax.cond(maybe, fixup, lambda: None)

        pltpu.sync_copy(rows_v, out_hbm.at[pl.ds(tb, CHUNK)])
        return carry

    lax.fori_loop(0, NCHUNK, chunk_body, 0)


def kernel(x, pretrained_embedding, residual_embedding, residual_index):
    y = _sc_lookup(x.reshape(N_TOK), pretrained_embedding,
                   residual_embedding, residual_index)
    return y.reshape(B, L, D)


# speedup vs baseline: 11.5408x; 11.5408x over previous
import functools

import jax
import jax.numpy as jnp
from jax import lax
from jax.experimental import pallas as pl
from jax.experimental.pallas import tpu as pltpu
from jax.experimental.pallas import tpu_sc as plsc

VOCAB = 100000
D = 64
B = 4096
L = 50
R = 128

N_TOK = B * L
CHUNK = 128
LANES = 16

_info = plsc.get_sparse_core_info()
NC, NS = _info.num_cores, _info.num_subcores
NW = NC * NS
TOK_PER_W = N_TOK // NW
NCHUNK = TOK_PER_W // CHUNK

_BS_STEPS = (64, 32, 16, 8, 4, 2, 1)

_mesh = plsc.VectorSubcoreMesh(core_axis_name="c", subcore_axis_name="s")


@functools.partial(
    pl.kernel,
    mesh=_mesh,
    out_type=jax.ShapeDtypeStruct((N_TOK, D), jnp.float32),
    compiler_params=pltpu.CompilerParams(needs_layout_passes=False,
                                         use_tc_tiling_on_sc=False),
    scratch_types=[
        pltpu.VMEM((CHUNK,), jnp.int32),
        pltpu.VMEM((CHUNK, D), jnp.float32),
        pltpu.VMEM((R,), jnp.int32),
        pltpu.VMEM((R, D), jnp.float32),
        pltpu.VMEM((LANES,), jnp.int32),
        pltpu.VMEM((LANES,), jnp.int32),
        pltpu.SemaphoreType.DMA,
    ],
)
def _sc_lookup(x_hbm, pre_hbm, res_hbm, ridx_hbm, out_hbm,
               idx_v, rows_v, ridx_v, rtab_v, tpos_v, tslot_v, sem):
    cid = lax.axis_index("c")
    sid = lax.axis_index("s")
    wid = sid * NC + cid
    base = wid * TOK_PER_W

    pltpu.sync_copy(ridx_hbm, ridx_v)
    pltpu.sync_copy(res_hbm, rtab_v)

    iota16 = lax.iota(jnp.int32, LANES)
    zero16 = jnp.zeros((LANES,), jnp.int32)
    rmin_s = jnp.min(plsc.load_gather(ridx_v, [zero16]))
    rmax_s = jnp.max(plsc.load_gather(ridx_v, [zero16 + (R - 1)]))

    def chunk_body(ci, carry):
        tb = base + ci * CHUNK
        pltpu.sync_copy(x_hbm.at[pl.ds(tb, CHUNK)], idx_v)
        pltpu.async_copy(pre_hbm.at[idx_v], rows_v, sem).wait()

        mn = jnp.full((LANES,), jnp.iinfo(jnp.int32).max, jnp.int32)
        mx = jnp.full((LANES,), jnp.iinfo(jnp.int32).min, jnp.int32)
        for i in range(CHUNK // LANES):
            v = idx_v[pl.ds(i * LANES, LANES)]
            mn = jnp.minimum(mn, v)
            mx = jnp.maximum(mx, v)
        maybe = (jnp.min(mn) <= rmax_s) & (jnp.max(mx) >= rmin_s)

        def fixup():
            for i in range(CHUNK // LANES):
                v = idx_v[pl.ds(i * LANES, LANES)]
                lo = zero16
                for sz in _BS_STEPS:
                    mid = lo + sz
                    probe = plsc.load_gather(ridx_v, [mid - 1])
                    lo = jnp.where(probe < v, mid, lo)
                safe = jnp.minimum(lo, R - 1)
                hit = (plsc.load_gather(ridx_v, [safe]) == v) & (lo < R)
                cnt = jnp.sum(hit.astype(jnp.int32))

                def do_fix():
                    plsc.store_compressed(tpos_v.at[...], i * LANES + iota16,
                                          mask=hit)
                    plsc.store_compressed(tslot_v.at[...], lo, mask=hit)

                    def fix_one(j, c):
                        jb = jnp.full((LANES,), j, jnp.int32)
                        p_b = plsc.load_gather(tpos_v, [jb])
                        s_b = plsc.load_gather(tslot_v, [jb])
                        for d in range(D // LANES):
                            colv = iota16 + d * LANES
                            vals = plsc.load_gather(rtab_v, [s_b, colv])
                            plsc.store_scatter(rows_v, [p_b, colv], vals)
                        return c

                    lax.fori_loop(0, cnt, fix_one, 0)

                lax.cond(cnt > 0, do_fix, lambda: None)

        lax.cond(maybe, fixup, lambda: None)

        pltpu.sync_copy(rows_v, out_hbm.at[pl.ds(tb, CHUNK)])
        return carry

    lax.fori_loop(0, NCHUNK, chunk_body, 0)


def kernel(x, pretrained_embedding, residual_embedding, residual_index):
    y = _sc_lookup(x.reshape(N_TOK), pretrained_embedding,
                   residual_embedding, residual_index)
    return y.reshape(B, L, D)


# speedup vs baseline: 14.4887x; 1.2554x over previous
import functools

import jax
import jax.numpy as jnp
from jax import lax
from jax.experimental import pallas as pl
from jax.experimental.pallas import tpu as pltpu
from jax.experimental.pallas import tpu_sc as plsc

VOCAB = 100000
D = 64
B = 4096
L = 50
R = 128

N_TOK = B * L
CHUNK = 128
LANES = 16
K = 5
SUP = K * CHUNK

_info = plsc.get_sparse_core_info()
NC, NS = _info.num_cores, _info.num_subcores
NW = NC * NS
TOK_PER_W = N_TOK // NW
NCHUNK = TOK_PER_W // CHUNK
NSUP = TOK_PER_W // SUP

_BS_STEPS = (64, 32, 16, 8, 4, 2, 1)

_mesh = plsc.VectorSubcoreMesh(core_axis_name="c", subcore_axis_name="s")


@functools.partial(
    pl.kernel,
    mesh=_mesh,
    out_type=jax.ShapeDtypeStruct((N_TOK, D), jnp.float32),
    compiler_params=pltpu.CompilerParams(needs_layout_passes=False,
                                         use_tc_tiling_on_sc=False),
    scratch_types=[
        pltpu.VMEM((NCHUNK, CHUNK), jnp.int32),
        pltpu.VMEM((SUP, D), jnp.float32),
        pltpu.VMEM((SUP, D), jnp.float32),
        pltpu.VMEM((R,), jnp.int32),
        pltpu.VMEM((R, D), jnp.float32),
        pltpu.VMEM((LANES,), jnp.int32),
        pltpu.VMEM((LANES,), jnp.int32),
        pltpu.SemaphoreType.DMA,
        pltpu.SemaphoreType.DMA,
        pltpu.SemaphoreType.DMA,
        pltpu.SemaphoreType.DMA,
    ],
)
def _sc_lookup(x_hbm, pre_hbm, res_hbm, ridx_hbm, out_hbm,
               idx_all, rows0, rows1, ridx_v, rtab_v, tpos_v, tslot_v,
               gsem0, gsem1, wsem0, wsem1):
    cid = lax.axis_index("c")
    sid = lax.axis_index("s")
    wid = sid * NC + cid
    wbase = wid * TOK_PER_W

    pltpu.sync_copy(x_hbm.at[pl.ds(wid * NCHUNK, NCHUNK)], idx_all)
    pltpu.sync_copy(ridx_hbm, ridx_v)
    pltpu.sync_copy(res_hbm, rtab_v)

    iota16 = lax.iota(jnp.int32, LANES)
    zero16 = jnp.zeros((LANES,), jnp.int32)
    rmin_s = jnp.min(plsc.load_gather(ridx_v, [zero16]))
    rmax_s = jnp.max(plsc.load_gather(ridx_v, [zero16 + (R - 1)]))

    rows = (rows0, rows1)
    gsem = (gsem0, gsem1)
    wsem = (wsem0, wsem1)

    def gather_copies(s, b):
        cbase = s * K
        return [
            pltpu.make_async_copy(
                pre_hbm.at[idx_all.at[cbase + j]],
                rows[b].at[pl.ds(j * CHUNK, CHUNK)],
                gsem[b],
            )
            for j in range(K)
        ]

    def fire_gathers(s, b):
        cbase = s * K
        for j in range(K):
            pltpu.async_copy(
                pre_hbm.at[idx_all.at[cbase + j]],
                rows[b].at[pl.ds(j * CHUNK, CHUNK)],
                gsem[b],
            )

    def wb_copy(s, b):
        return pltpu.make_async_copy(
            rows[b], out_hbm.at[pl.ds(wbase + s * SUP, SUP)], wsem[b])

    def fixup(s, b):
        cbase = s * K
        rows_b = rows[b]
        for j in range(K):
            crow = idx_all.at[cbase + j]
            mn = jnp.full((LANES,), jnp.iinfo(jnp.int32).max, jnp.int32)
            mx = jnp.full((LANES,), jnp.iinfo(jnp.int32).min, jnp.int32)
            for i in range(CHUNK // LANES):
                v = crow[pl.ds(i * LANES, LANES)]
                mn = jnp.minimum(mn, v)
                mx = jnp.maximum(mx, v)
            maybe = (jnp.min(mn) <= rmax_s) & (jnp.max(mx) >= rmin_s)

            def fine(j=j, crow=crow):
                def vec_body(i, carry):
                    v = crow[pl.ds(i * LANES, LANES)]
                    lo = zero16
                    for sz in _BS_STEPS:
                        mid = lo + sz
                        probe = plsc.load_gather(ridx_v, [mid - 1])
                        lo = jnp.where(probe < v, mid, lo)
                    safe = jnp.minimum(lo, R - 1)
                    hit = (plsc.load_gather(ridx_v, [safe]) == v) & (lo < R)
                    cnt = jnp.sum(hit.astype(jnp.int32))

                    def do_fix():
                        plsc.store_compressed(
                            tpos_v.at[...], j * CHUNK + i * LANES + iota16,
                            mask=hit)
                        plsc.store_compressed(tslot_v.at[...], lo, mask=hit)

                        def fix_one(q, c):
                            qb = jnp.full((LANES,), q, jnp.int32)
                            p_b = plsc.load_gather(tpos_v, [qb])
                            s_b = plsc.load_gather(tslot_v, [qb])
                            for d in range(D // LANES):
                                colv = iota16 + d * LANES
                                vals = plsc.load_gather(rtab_v, [s_b, colv])
                                plsc.store_scatter(rows_b, [p_b, colv], vals)
                            return c

                        lax.fori_loop(0, cnt, fix_one, 0)

                    lax.cond(cnt > 0, do_fix, lambda: None)
                    return carry

                lax.fori_loop(0, CHUNK // LANES, vec_body, 0)

            lax.cond(maybe, fine, lambda: None)

    fire_gathers(0, 0)
    fire_gathers(1, 1)

    def super_body(s2, carry):
        for b in range(2):
            s = 2 * s2 + b
            for cp in gather_copies(s, b):
                cp.wait()
            fixup(s, b)
            wb = wb_copy(s, b)
            wb.start()

            def fire_next(s=s, b=b, wb=wb):
                wb.wait()
                fire_gathers(s + 2, b)

            lax.cond(s + 2 < NSUP, fire_next, lambda: None)
        return carry

    lax.fori_loop(0, NSUP // 2, super_body, 0)

    wb_copy(NSUP - 2, 0).wait()
    wb_copy(NSUP - 1, 1).wait()


def kernel(x, pretrained_embedding, residual_embedding, residual_index):
    y = _sc_lookup(x.reshape(N_TOK // CHUNK, CHUNK), pretrained_embedding,
                   residual_embedding, residual_index)
    return y.reshape(B, L, D)
